# baseline (speedup 1.0000x reference)
import functools

import jax
import jax.numpy as jnp
import numpy as np
from jax import lax
from jax.experimental import pallas as pl
from jax.experimental.pallas import tpu as pltpu
from jax.experimental.pallas import tpu_sc as plsc

SN_LEN = 3
KSAMP = 64
RADIUS = 1.0
B, N, M = 4, 10000, 1024
DESC = 128

NP = 10240
NV = NP // 16
NW = 32
RPW = M // NW
R2BITS = 0x3F800000
INFBITS = 0x7F800000
PAD_XYZ = 1e18

_PERM_CACHE = None


def _perm_const(x):
    global _PERM_CACHE
    if _PERM_CACHE is None:
        try:
            _PERM_CACHE = np.asarray(
                jax.random.permutation(jax.random.key(42), N))
        except Exception:
            kd = (jnp.zeros((2,), jnp.uint32).at[1].set(42)
                  + (0 * x[0, 0, 0]).astype(jnp.uint32))
            key = jax.random.wrap_key_data(kd, impl="threefry2x32")
            return jax.random.permutation(key, N)
    return jnp.asarray(_PERM_CACHE)

_BITONIC_LAYERS = []
for _k in (2, 4, 8, 16, 32, 64):
    _j = _k // 2
    while _j >= 1:
        _BITONIC_LAYERS.append((_k, _j))
        _j //= 2

_LANE = np.arange(16, dtype=np.int32)


def _want_min_mask(k, j, v):
    i = v * 16 + _LANE
    asc = (i & k) == 0
    lower = (i & j) == 0
    return np.asarray(asc == lower)


def _lex_smaller(bu, bi, au, ai):
    return (bu < au) | ((bu == au) & (bi < ai))


_GDN = jax.lax.GatherDimensionNumbers(
    offset_dims=(), collapsed_slice_dims=(0,), start_index_map=(0,))


def _take16(x, idx):
    return jax.lax.gather(x, idx[:, None], _GDN, (1,),
                          mode=jax.lax.GatherScatterMode.PROMISE_IN_BOUNDS)


def _bitonic_sort64(us, ids, iota=None):
    us = list(us)
    ids = list(ids)
    if iota is None:
        iota = jnp.asarray(_LANE)
    for (k, j) in _BITONIC_LAYERS:
        if j >= 16:
            vs = j // 16
            for va in range(4):
                if va & vs:
                    continue
                vb = va | vs
                asc = ((va * 16) & k) == 0
                au, ai, bu, bi = us[va], ids[va], us[vb], ids[vb]
                bs = _lex_smaller(bu, bi, au, ai)
                if asc:
                    us[va] = jnp.where(bs, bu, au)
                    ids[va] = jnp.where(bs, bi, ai)
                    us[vb] = jnp.where(bs, au, bu)
                    ids[vb] = jnp.where(bs, ai, bi)
                else:
                    us[va] = jnp.where(bs, au, bu)
                    ids[va] = jnp.where(bs, ai, bi)
                    us[vb] = jnp.where(bs, bu, au)
                    ids[vb] = jnp.where(bs, bi, ai)
        else:
            perm = iota ^ j
            for v in range(4):
                iv = iota + v * 16
                wm = ((iv & k) == 0) == ((iv & j) == 0)
                u, i = us[v], ids[v]
                pu = _take16(u, perm)
                pi = _take16(i, perm)
                ps = _lex_smaller(pu, pi, u, i)
                selp = wm == ps
                us[v] = jnp.where(selp, pu, u)
                ids[v] = jnp.where(selp, pi, i)
    return us, ids


def _sc_ballquery(pts, kps, table, nb):
    mesh = plsc.VectorSubcoreMesh(core_axis_name="c", subcore_axis_name="s")

    i32 = jnp.int32
    f32 = jnp.float32

    @functools.partial(
        pl.kernel,
        out_type=[
            jax.ShapeDtypeStruct((nb, 6, M, KSAMP), f32),
            jax.ShapeDtypeStruct((nb, M * KSAMP * 8), f32),
        ],
        mesh=mesh,
        scratch_types=[
            pltpu.VMEM((3, NP), f32),
            pltpu.VMEM((128,), f32),
            pltpu.VMEM((NP + 128,), i32),
            pltpu.VMEM((NP + 128,), i32),
            pltpu.VMEM((256,), i32),
            pltpu.VMEM((256,), i32),
            pltpu.VMEM((128,), i32),
            pltpu.VMEM((128,), i32),
            pltpu.VMEM((KSAMP,), i32),
            pltpu.VMEM((KSAMP, 16), f32),
            pltpu.VMEM((6, RPW, KSAMP), f32),
            pltpu.VMEM((RPW * KSAMP * 8,), f32),
            pltpu.SemaphoreType.DMA,
        ],
        compiler_params=pltpu.CompilerParams(needs_layout_passes=False,
                                             use_tc_tiling_on_sc=False),
    )
    def ballquery(pts_hbm, kps_hbm, table_hbm, xf_hbm, xf2_hbm,
                  pts_v, kp_v, cand_u, cand_idx, hist, cumb,
                  top_u, top_idx, idxb, gath, xf_blk, xf2_blk, sem):
        cid = lax.axis_index("c")
        sid = lax.axis_index("s")
        wid = sid * 2 + cid
        m0 = wid * RPW

        iota = lax.iota(i32, 16)
        zeros16 = jnp.zeros((16,), i32)
        ones16 = jnp.ones((16,), i32)
        r2v = jnp.full((16,), R2BITS, i32)
        infv = jnp.full((16,), INFBITS, i32)
        splat0 = jnp.zeros((16,), i32)
        splat15 = jnp.full((16,), 15, i32)

        def count16(m):
            return plsc.all_reduce_population_count(m)

        def row_body(rb, b, _):
            kx = plsc.load_gather(kp_v, [jnp.full((16,), rb, i32)])
            ky = plsc.load_gather(kp_v, [jnp.full((16,), RPW + rb, i32)])
            kz = plsc.load_gather(kp_v, [jnp.full((16,), 2 * RPW + rb, i32)])

            @plsc.parallel_loop(0, NV, unroll=8, carry=zeros16)
            def offv(j, offv):
                j16 = j * 16
                px = pts_v[0, pl.ds(j16, 16)]
                py = pts_v[1, pl.ds(j16, 16)]
                pz = pts_v[2, pl.ds(j16, 16)]
                dx = px - kx
                dy = py - ky
                dz = pz - kz
                d2 = dx * dx + dy * dy + dz * dz
                uv = plsc.bitcast(d2, i32)
                m = uv < r2v
                rk = plsc.cumsum(jnp.where(m, 1, 0))
                pos = jnp.maximum(offv + rk - 1, 0)
                ivv = iota + jnp.full((16,), j16, i32)
                plsc.store_scatter(cand_u, [pos], uv, mask=m)
                plsc.store_scatter(cand_idx, [pos], ivv, mask=m)
                return offv + count16(m)
            n = jnp.max(offv)
            n_vec = jnp.full((16,), n, i32)
            nvr = (n + 15) >> 4

            def refine():
                v64 = zeros16
                need = jnp.full((16,), KSAMP, i32)
                for li, shift in enumerate((24, 16, 8, 0)):
                    for v in range(16):
                        hist[pl.ds(v * 16, 16)] = zeros16

                    v64c = v64

                    @plsc.parallel_loop(0, nvr, unroll=4)
                    def _(j, shift=shift, li=li, v64c=v64c):
                        j16 = j * 16
                        uv = cand_u[pl.ds(j16, 16)]
                        lanev = iota + jnp.full((16,), j16, i32)
                        mv = lanev < n_vec
                        if li > 0:
                            mv = mv & ((uv >> (shift + 8)) == (v64c >> (shift + 8)))
                        bk = (uv >> shift) & 0xFF
                        plsc.addupdate_scatter(hist, [bk], ones16, mask=mv)

                    carry = zeros16
                    tcnt = zeros16
                    for v in range(16):
                        h = hist[pl.ds(v * 16, 16)]
                        cc = plsc.cumsum(h) + carry
                        cumb[pl.ds(v * 16, 16)] = cc
                        carry = _take16(cc, splat15)
                        tcnt = tcnt + jnp.where(cc < need, 1, 0)
                    tv = jnp.full((16,), jnp.sum(tcnt), i32)
                    cum_t = plsc.load_gather(cumb, [tv])
                    h_t = plsc.load_gather(hist, [tv])
                    need = need - (cum_t - h_t)
                    v64 = v64 | (tv << shift)
                return v64, need

            v64, need = lax.cond(
                n > KSAMP, refine,
                lambda: (infv, zeros16))

            for v in range(4):
                top_u[pl.ds(v * 16, 16)] = infv
                top_idx[pl.ds(v * 16, 16)] = zeros16

            @plsc.parallel_loop(0, nvr, unroll=4, carry=(zeros16, zeros16))
            def _collected(j, carry):
                offv, tiev = carry
                j16 = j * 16
                uv = cand_u[pl.ds(j16, 16)]
                iv = cand_idx[pl.ds(j16, 16)]
                lanev = iota + jnp.full((16,), j16, i32)
                mvld = lanev < n_vec
                mlt = mvld & (uv < v64)
                meq = mvld & (uv == v64)
                req = plsc.cumsum(jnp.where(meq, 1, 0))
                seleq = meq & ((tiev + req) <= need)
                msel = mlt | seleq
                rsel = plsc.cumsum(jnp.where(msel, 1, 0))
                pos = jnp.maximum(offv + rsel - 1, 0)
                plsc.store_scatter(top_u, [pos], uv, mask=msel)
                plsc.store_scatter(top_idx, [pos], iv, mask=msel)
                return offv + count16(msel), tiev + count16(seleq)

            us = [top_u[pl.ds(v * 16, 16)] for v in range(4)]
            ids = [top_idx[pl.ds(v * 16, 16)] for v in range(4)]
            us, ids = _bitonic_sort64(us, ids, iota)
            nsel = jnp.full((16,), jnp.minimum(n, KSAMP), i32)
            first = _take16(ids[0], splat0)
            boff = b * NP
            for v in range(4):
                posv = iota + v * 16
                idv = jnp.where(posv >= nsel, first, ids[v])
                idxb[pl.ds(v * 16, 16)] = idv + boff

            pltpu.async_copy(table_hbm.at[idxb], gath, sem).wait()
            kcs = (kx, ky, kz)
            for c in range(6):
                colv = jnp.full((16,), c, i32)
                for v in range(4):
                    rows = iota + v * 16
                    vals = plsc.load_gather(gath, [rows, colv])
                    if c < 3:
                        vals = vals - kcs[c]
                    xf_blk[c, rb, pl.ds(v * 16, 16)] = vals
                    fpos = (jnp.full((16,), rb * KSAMP * 8, i32)
                            + (iota + v * 16) * 8 + c)
                    plsc.store_scatter(xf2_blk, [fpos], vals)
            return 0

        def b_body(b, _):
            pltpu.sync_copy(pts_hbm.at[b], pts_v)
            for c in range(3):
                pltpu.sync_copy(kps_hbm.at[b, c, pl.ds(m0, RPW)],
                                kp_v.at[pl.ds(c * RPW, RPW)])
            lax.fori_loop(0, RPW, lambda rb, x: row_body(rb, b, x), 0)
            for c in range(6):
                pltpu.sync_copy(xf_blk.at[c], xf_hbm.at[b, c, pl.ds(m0, RPW)])
            pltpu.sync_copy(xf2_blk,
                            xf2_hbm.at[b, pl.ds(m0 * KSAMP * 8,
                                                RPW * KSAMP * 8)])
            return 0

        lax.fori_loop(0, nb, b_body, 0)

    return ballquery(pts, kps, table)


MB = 64


def _mlp_block(xf_ref, w1_ref, b1_ref, w2_ref, b2_ref, w3_ref, b3_ref,
               w4_ref, b4_ref, w5_ref, b5_ref, out_ref):
    x = xf_ref[0]
    y = jnp.maximum(
        jax.lax.dot_general(x, w1_ref[...], (((1,), (0,)), ((), ())),
                            preferred_element_type=jnp.float32) + b1_ref[...],
        0.0)
    y = jnp.maximum(
        jax.lax.dot_general(y, w2_ref[...], (((1,), (0,)), ((), ())),
                            preferred_element_type=jnp.float32) + b2_ref[...],
        0.0)
    y3 = jnp.maximum(
        jax.lax.dot_general(y, w3_ref[...], (((1,), (0,)), ((), ())),
                            preferred_element_type=jnp.float32) + b3_ref[...],
        0.0)
    y3r = y3.reshape(MB, KSAMP, DESC)
    y3m = jnp.max(y3r, axis=1, keepdims=True)
    y3mb = jnp.broadcast_to(y3m, (MB, KSAMP, DESC)).reshape(MB * KSAMP, DESC)
    cat = jnp.concatenate([y3, y3mb], axis=1)
    y4 = jnp.maximum(
        jax.lax.dot_general(cat, w4_ref[...], (((1,), (0,)), ((), ())),
                            preferred_element_type=jnp.float32) + b4_ref[...],
        0.0)
    y5 = jax.lax.dot_general(y4, w5_ref[...], (((1,), (0,)), ((), ())),
                             preferred_element_type=jnp.float32) + b5_ref[...]
    desc = jnp.max(y5.reshape(MB, KSAMP, DESC), axis=1)
    nrm = jnp.sqrt(jnp.sum(desc * desc, axis=1, keepdims=True)) + 1e-5
    out_ref[0] = desc / nrm


def _run_mlp(xf2, nb, w1, b1, w2, b2, w3, b3, w4, b4, w5, b5):
    grid = (nb, M // MB)
    P = MB * KSAMP

    def w_spec(shape):
        return pl.BlockSpec(shape, lambda b, m: (0,) * len(shape))

    return pl.pallas_call(
        _mlp_block,
        grid=grid,
        in_specs=[
            pl.BlockSpec((1, P, 8), lambda b, m: (b, m, 0)),
            w_spec(w1.shape), w_spec(b1.shape),
            w_spec(w2.shape), w_spec(b2.shape),
            w_spec(w3.shape), w_spec(b3.shape),
            w_spec(w4.shape), w_spec(b4.shape),
            w_spec(w5.shape), w_spec(b5.shape),
        ],
        out_specs=pl.BlockSpec((1, MB, DESC), lambda b, m: (b, m, 0)),
        out_shape=jax.ShapeDtypeStruct((nb, M, DESC), jnp.float32),
    )(xf2, w1, b1, w2, b2, w3, b3, w4, b4, w5, b5)


def kernel(x, sn, keypoints, W1, b1, W2, b2, W3, b3, W4, b4, W5, b5,
           g1, beta1, g2, beta2, g3, beta3, g4, beta4):
    perm = _perm_const(x)
    xp = x[:, :, perm]
    snp = sn[:, :, perm]

    pts = jnp.pad(xp, ((0, 0), (0, 0), (0, NP - N)),
                  constant_values=PAD_XYZ)
    xa = jnp.concatenate([xp, snp], axis=1)
    table = jnp.pad(jnp.transpose(xa, (0, 2, 1)),
                    ((0, 0), (0, NP - N), (0, 10)))
    table = table.reshape(B * NP, 16)

    h = B // 2
    table = table.reshape(B, NP, 16)
    xf_a, xf2_a = _sc_ballquery(pts[:h], keypoints[:, :, :][:h],
                                table[:h].reshape(h * NP, 16), h)
    xf_b, xf2_b = _sc_ballquery(pts[h:], keypoints[h:],
                                table[h:].reshape(h * NP, 16), h)

    s = 1.0 / np.sqrt(1.0 + 1e-5)
    w1f = (W1 * (s * g1)[:, None]).T
    b1f = b1 * s * g1 + beta1
    w2f = (W2 * (s * g2)[:, None]).T
    b2f = b2 * s * g2 + beta2
    w3f = (W3 * (s * g3)[:, None]).T
    b3f = b3 * s * g3 + beta3
    w4f = (W4 * (s * g4)[:, None]).T
    b4f = b4 * s * g4 + beta4
    w5f = W5.T
    b5f = b5
    w1f = jnp.pad(w1f, ((0, 2), (0, 0)))

    desc_a = _run_mlp(xf2_a.reshape(h, M * KSAMP, 8), h,
                      w1f, b1f, w2f, b2f, w3f, b3f, w4f, b4f, w5f, b5f)
    desc_b = _run_mlp(xf2_b.reshape(h, M * KSAMP, 8), h,
                      w1f, b1f, w2f, b2f, w3f, b3f, w4f, b4f, w5f, b5f)
    desc = jnp.concatenate([desc_a, desc_b], axis=0)
    x_features = jnp.concatenate([xf_a, xf_b], axis=0)
    descriptor = jnp.transpose(desc, (0, 2, 1))
    return (descriptor, x_features)


# speedup vs baseline: 1.0081x; 1.0081x over previous
import functools

import jax
import jax.numpy as jnp
import numpy as np
from jax import lax
from jax.experimental import pallas as pl
from jax.experimental.pallas import tpu as pltpu
from jax.experimental.pallas import tpu_sc as plsc

SN_LEN = 3
KSAMP = 64
RADIUS = 1.0
B, N, M = 4, 10000, 1024
DESC = 128

NP = 10240
NV = NP // 16
NW = 32
RPW = M // NW
R2BITS = 0x3F800000
INFBITS = 0x7F800000
PAD_XYZ = 1e18

_PERM_CACHE = None


def _perm_const(x):
    global _PERM_CACHE
    if _PERM_CACHE is None:
        try:
            _PERM_CACHE = np.asarray(
                jax.random.permutation(jax.random.key(42), N))
        except Exception:
            kd = (jnp.zeros((2,), jnp.uint32).at[1].set(42)
                  + (0 * x[0, 0, 0]).astype(jnp.uint32))
            key = jax.random.wrap_key_data(kd, impl="threefry2x32")
            return jax.random.permutation(key, N)
    return jnp.asarray(_PERM_CACHE)

_BITONIC_LAYERS = []
for _k in (2, 4, 8, 16, 32, 64):
    _j = _k // 2
    while _j >= 1:
        _BITONIC_LAYERS.append((_k, _j))
        _j //= 2

_LANE = np.arange(16, dtype=np.int32)


def _want_min_mask(k, j, v):
    i = v * 16 + _LANE
    asc = (i & k) == 0
    lower = (i & j) == 0
    return np.asarray(asc == lower)


def _lex_smaller(bu, bi, au, ai):
    return (bu < au) | ((bu == au) & (bi < ai))


_GDN = jax.lax.GatherDimensionNumbers(
    offset_dims=(), collapsed_slice_dims=(0,), start_index_map=(0,))


def _take16(x, idx):
    return jax.lax.gather(x, idx[:, None], _GDN, (1,),
                          mode=jax.lax.GatherScatterMode.PROMISE_IN_BOUNDS)


def _bitonic_sort64(us, ids, iota=None):
    us = list(us)
    ids = list(ids)
    if iota is None:
        iota = jnp.asarray(_LANE)
    for (k, j) in _BITONIC_LAYERS:
        if j >= 16:
            vs = j // 16
            for va in range(4):
                if va & vs:
                    continue
                vb = va | vs
                asc = ((va * 16) & k) == 0
                au, ai, bu, bi = us[va], ids[va], us[vb], ids[vb]
                bs = _lex_smaller(bu, bi, au, ai)
                if asc:
                    us[va] = jnp.where(bs, bu, au)
                    ids[va] = jnp.where(bs, bi, ai)
                    us[vb] = jnp.where(bs, au, bu)
                    ids[vb] = jnp.where(bs, ai, bi)
                else:
                    us[va] = jnp.where(bs, au, bu)
                    ids[va] = jnp.where(bs, ai, bi)
                    us[vb] = jnp.where(bs, bu, au)
                    ids[vb] = jnp.where(bs, bi, ai)
        else:
            perm = iota ^ j
            for v in range(4):
                iv = iota + v * 16
                wm = ((iv & k) == 0) == ((iv & j) == 0)
                u, i = us[v], ids[v]
                pu = _take16(u, perm)
                pi = _take16(i, perm)
                ps = _lex_smaller(pu, pi, u, i)
                selp = wm == ps
                us[v] = jnp.where(selp, pu, u)
                ids[v] = jnp.where(selp, pi, i)
    return us, ids


def _sc_ballquery(pts, kps, table, nb):
    mesh = plsc.VectorSubcoreMesh(core_axis_name="c", subcore_axis_name="s")

    i32 = jnp.int32
    f32 = jnp.float32

    @functools.partial(
        pl.kernel,
        out_type=[
            jax.ShapeDtypeStruct((nb, 6, M, KSAMP), f32),
            jax.ShapeDtypeStruct((nb, M * KSAMP * 8), f32),
        ],
        mesh=mesh,
        scratch_types=[
            pltpu.VMEM((3, NP), f32),
            pltpu.VMEM((128,), f32),
            pltpu.VMEM((NP + 128,), i32),
            pltpu.VMEM((NP + 128,), i32),
            pltpu.VMEM((256,), i32),
            pltpu.VMEM((256,), i32),
            pltpu.VMEM((128,), i32),
            pltpu.VMEM((128,), i32),
            pltpu.VMEM((KSAMP,), i32),
            pltpu.VMEM((KSAMP, 16), f32),
            pltpu.VMEM((6, RPW, KSAMP), f32),
            pltpu.VMEM((RPW * KSAMP * 8,), f32),
            pltpu.SemaphoreType.DMA,
        ],
        compiler_params=pltpu.CompilerParams(needs_layout_passes=False,
                                             use_tc_tiling_on_sc=False),
    )
    def ballquery(pts_hbm, kps_hbm, table_hbm, xf_hbm, xf2_hbm,
                  pts_v, kp_v, cand_u, cand_idx,
                  hist, cumb, top_u, top_idx, idxb, gath, xf_blk, xf2_blk,
                  sem):
        cid = lax.axis_index("c")
        sid = lax.axis_index("s")
        wid = sid * 2 + cid
        m0 = wid * RPW

        iota = lax.iota(i32, 16)
        zeros16 = jnp.zeros((16,), i32)
        ones16 = jnp.ones((16,), i32)
        r2v = jnp.full((16,), R2BITS, i32)
        infv = jnp.full((16,), INFBITS, i32)
        splat0 = jnp.zeros((16,), i32)
        splat15 = jnp.full((16,), 15, i32)

        def count16(m):
            return plsc.all_reduce_population_count(m)

        def emit_row(rb, b, kx, ky, kz, cu, ci_, offv):
            n = jnp.max(offv)
            n_vec = jnp.full((16,), n, i32)
            nvr = (n + 15) >> 4

            def refine():
                v64 = zeros16
                need = jnp.full((16,), KSAMP, i32)
                for li, shift in enumerate((24, 16, 8, 0)):
                    for v in range(16):
                        hist[pl.ds(v * 16, 16)] = zeros16

                    v64c = v64

                    @plsc.parallel_loop(0, nvr, unroll=4)
                    def _(j, shift=shift, li=li, v64c=v64c):
                        j16 = j * 16
                        uv = cu[pl.ds(j16, 16)]
                        lanev = iota + jnp.full((16,), j16, i32)
                        mv = lanev < n_vec
                        if li > 0:
                            mv = mv & ((uv >> (shift + 8)) == (v64c >> (shift + 8)))
                        bk = (uv >> shift) & 0xFF
                        plsc.addupdate_scatter(hist, [bk], ones16, mask=mv)

                    carry = zeros16
                    tcnt = zeros16
                    for v in range(16):
                        h = hist[pl.ds(v * 16, 16)]
                        cc = plsc.cumsum(h) + carry
                        cumb[pl.ds(v * 16, 16)] = cc
                        carry = _take16(cc, splat15)
                        tcnt = tcnt + jnp.where(cc < need, 1, 0)
                    tv = jnp.full((16,), jnp.sum(tcnt), i32)
                    cum_t = plsc.load_gather(cumb, [tv])
                    h_t = plsc.load_gather(hist, [tv])
                    need = need - (cum_t - h_t)
                    v64 = v64 | (tv << shift)
                return v64, need

            v64, need = lax.cond(
                n > KSAMP, refine,
                lambda: (infv, zeros16))

            for v in range(4):
                top_u[pl.ds(v * 16, 16)] = infv
                top_idx[pl.ds(v * 16, 16)] = zeros16

            @plsc.parallel_loop(0, nvr, unroll=4, carry=(zeros16, zeros16))
            def _collected(j, carry):
                offv, tiev = carry
                j16 = j * 16
                uv = cu[pl.ds(j16, 16)]
                iv = ci_[pl.ds(j16, 16)]
                lanev = iota + jnp.full((16,), j16, i32)
                mvld = lanev < n_vec
                mlt = mvld & (uv < v64)
                meq = mvld & (uv == v64)
                req = plsc.cumsum(jnp.where(meq, 1, 0))
                seleq = meq & ((tiev + req) <= need)
                msel = mlt | seleq
                rsel = plsc.cumsum(jnp.where(msel, 1, 0))
                pos = jnp.maximum(offv + rsel - 1, 0)
                plsc.store_scatter(top_u, [pos], uv, mask=msel)
                plsc.store_scatter(top_idx, [pos], iv, mask=msel)
                return offv + count16(msel), tiev + count16(seleq)

            us = [top_u[pl.ds(v * 16, 16)] for v in range(4)]
            ids = [top_idx[pl.ds(v * 16, 16)] for v in range(4)]
            us, ids = _bitonic_sort64(us, ids, iota)
            nsel = jnp.full((16,), jnp.minimum(n, KSAMP), i32)
            first = _take16(ids[0], splat0)
            boff = b * NP
            for v in range(4):
                posv = iota + v * 16
                idv = jnp.where(posv >= nsel, first, ids[v])
                idxb[pl.ds(v * 16, 16)] = idv + boff

            pltpu.async_copy(table_hbm.at[idxb], gath, sem).wait()
            kcs = (kx, ky, kz)
            for c in range(6):
                colv = jnp.full((16,), c, i32)
                for v in range(4):
                    rows = iota + v * 16
                    vals = plsc.load_gather(gath, [rows, colv])
                    if c < 3:
                        vals = vals - kcs[c]
                    xf_blk[c, rb, pl.ds(v * 16, 16)] = vals
                    fpos = (jnp.full((16,), rb * KSAMP * 8, i32)
                            + (iota + v * 16) * 8 + c)
                    plsc.store_scatter(xf2_blk, [fpos], vals)
            return 0

        def row_body(rb, b, _):
            def kvec(r, c):
                return plsc.load_gather(
                    kp_v, [jnp.full((16,), c * RPW + r, i32)])

            kx, ky, kz = kvec(rb, 0), kvec(rb, 1), kvec(rb, 2)

            @plsc.parallel_loop(0, NV, unroll=8, carry=zeros16)
            def offv(j, offv):
                j16 = j * 16
                px = pts_v[0, pl.ds(j16, 16)]
                py = pts_v[1, pl.ds(j16, 16)]
                pz = pts_v[2, pl.ds(j16, 16)]
                dx = px - kx
                dy = py - ky
                dz = pz - kz
                d2 = dx * dx + dy * dy + dz * dz
                uv = plsc.bitcast(d2, i32)
                m = uv < r2v
                rk = plsc.cumsum(jnp.where(m, 1, 0))
                pos = jnp.maximum(offv + rk - 1, 0)
                ivv = iota + jnp.full((16,), j16, i32)
                plsc.store_scatter(cand_u, [pos], uv, mask=m)
                plsc.store_scatter(cand_idx, [pos], ivv, mask=m)
                return offv + count16(m)

            emit_row(rb, b, kx, ky, kz, cand_u, cand_idx, offv)
            return 0

        def b_body(b, _):
            pltpu.sync_copy(pts_hbm.at[b], pts_v)
            for c in range(3):
                pltpu.sync_copy(kps_hbm.at[b, c, pl.ds(m0, RPW)],
                                kp_v.at[pl.ds(c * RPW, RPW)])
            lax.fori_loop(0, RPW, lambda rb, x: row_body(rb, b, x), 0)
            for c in range(6):
                pltpu.sync_copy(xf_blk.at[c], xf_hbm.at[b, c, pl.ds(m0, RPW)])
            pltpu.sync_copy(xf2_blk,
                            xf2_hbm.at[b, pl.ds(m0 * KSAMP * 8,
                                                RPW * KSAMP * 8)])
            return 0

        lax.fori_loop(0, nb, b_body, 0)

    return ballquery(pts, kps, table)


MB = 64


def _mlp_block(xf_ref, w1_ref, b1_ref, w2_ref, b2_ref, w3_ref, b3_ref,
               w4_ref, b4_ref, w5_ref, b5_ref, out_ref):
    x = xf_ref[0]
    y = jnp.maximum(
        jax.lax.dot_general(x, w1_ref[...], (((1,), (0,)), ((), ())),
                            preferred_element_type=jnp.float32) + b1_ref[...],
        0.0)
    y = jnp.maximum(
        jax.lax.dot_general(y, w2_ref[...], (((1,), (0,)), ((), ())),
                            preferred_element_type=jnp.float32) + b2_ref[...],
        0.0)
    y3 = jnp.maximum(
        jax.lax.dot_general(y, w3_ref[...], (((1,), (0,)), ((), ())),
                            preferred_element_type=jnp.float32) + b3_ref[...],
        0.0)
    y3r = y3.reshape(MB, KSAMP, DESC)
    y3m = jnp.max(y3r, axis=1, keepdims=True)
    y3mb = jnp.broadcast_to(y3m, (MB, KSAMP, DESC)).reshape(MB * KSAMP, DESC)
    cat = jnp.concatenate([y3, y3mb], axis=1)
    y4 = jnp.maximum(
        jax.lax.dot_general(cat, w4_ref[...], (((1,), (0,)), ((), ())),
                            preferred_element_type=jnp.float32) + b4_ref[...],
        0.0)
    y5 = jax.lax.dot_general(y4, w5_ref[...], (((1,), (0,)), ((), ())),
                             preferred_element_type=jnp.float32) + b5_ref[...]
    desc = jnp.max(y5.reshape(MB, KSAMP, DESC), axis=1)
    nrm = jnp.sqrt(jnp.sum(desc * desc, axis=1, keepdims=True)) + 1e-5
    out_ref[0] = desc / nrm


def _run_mlp(xf2, nb, w1, b1, w2, b2, w3, b3, w4, b4, w5, b5):
    grid = (nb, M // MB)
    P = MB * KSAMP

    def w_spec(shape):
        return pl.BlockSpec(shape, lambda b, m: (0,) * len(shape))

    return pl.pallas_call(
        _mlp_block,
        grid=grid,
        in_specs=[
            pl.BlockSpec((1, P, 8), lambda b, m: (b, m, 0)),
            w_spec(w1.shape), w_spec(b1.shape),
            w_spec(w2.shape), w_spec(b2.shape),
            w_spec(w3.shape), w_spec(b3.shape),
            w_spec(w4.shape), w_spec(b4.shape),
            w_spec(w5.shape), w_spec(b5.shape),
        ],
        out_specs=pl.BlockSpec((1, MB, DESC), lambda b, m: (b, m, 0)),
        out_shape=jax.ShapeDtypeStruct((nb, M, DESC), jnp.float32),
    )(xf2, w1, b1, w2, b2, w3, b3, w4, b4, w5, b5)


def kernel(x, sn, keypoints, W1, b1, W2, b2, W3, b3, W4, b4, W5, b5,
           g1, beta1, g2, beta2, g3, beta3, g4, beta4):
    perm = _perm_const(x)
    xp = x[:, :, perm]
    snp = sn[:, :, perm]

    pts = jnp.pad(xp, ((0, 0), (0, 0), (0, NP - N)),
                  constant_values=PAD_XYZ)
    xa = jnp.concatenate([xp, snp], axis=1)
    table = jnp.pad(jnp.transpose(xa, (0, 2, 1)),
                    ((0, 0), (0, NP - N), (0, 10)))
    table = table.reshape(B * NP, 16)

    x_features, xf2_flat = _sc_ballquery(pts, keypoints, table, B)

    s = 1.0 / np.sqrt(1.0 + 1e-5)
    w1f = (W1 * (s * g1)[:, None]).T
    b1f = b1 * s * g1 + beta1
    w2f = (W2 * (s * g2)[:, None]).T
    b2f = b2 * s * g2 + beta2
    w3f = (W3 * (s * g3)[:, None]).T
    b3f = b3 * s * g3 + beta3
    w4f = (W4 * (s * g4)[:, None]).T
    b4f = b4 * s * g4 + beta4
    w5f = W5.T
    b5f = b5
    w1f = jnp.pad(w1f, ((0, 2), (0, 0)))

    desc = _run_mlp(xf2_flat.reshape(B, M * KSAMP, 8), B,
                    w1f, b1f, w2f, b2f, w3f, b3f, w4f, b4f, w5f, b5f)
    descriptor = jnp.transpose(desc, (0, 2, 1))
    return (descriptor, x_features)
